# Initial kernel scaffold; baseline (speedup 1.0000x reference)
import jax
import jax.numpy as jnp
from jax.experimental import pallas as pl


def kernel(u, x, weights):
    raise NotImplementedError("write your pallas kernel here")


# speedup vs baseline: 276.4393x; 276.4393x over previous
import jax
import jax.numpy as jnp
from jax.experimental import pallas as pl


def _soft_threshold_kernel(u_ref, x_ref, w_ref, o_ref):
    u = u_ref[...]
    x = x_ref[...]
    w = w_ref[0, 0]
    b = u - x
    mag = jnp.maximum(jnp.abs(b) - w, 0.0)
    o_ref[...] = x + jnp.sign(b) * mag


def kernel(u, x, weights):
    p = u.shape[0]
    u2 = u.reshape(8, p // 8)
    x2 = x.reshape(8, p // 8)
    w2 = jnp.reshape(weights, (1, 1)).astype(u.dtype)
    out = pl.pallas_call(
        _soft_threshold_kernel,
        out_shape=jax.ShapeDtypeStruct(u2.shape, u.dtype),
    )(u2, x2, w2)
    return out.reshape(x.shape)
